# baseline (speedup 1.0000x reference)
import functools

import jax
import jax.numpy as jnp
from jax.experimental import pallas as pl
from jax.experimental.pallas import tpu as pltpu

N = 10000
D_IN = 128
HID = 32
D_OUT = 16
NCHUNK = 8
CHUNK = N // NCHUNK


def _mlp(x_hbm, w0_ref, b0_ref, w1_ref, b1_ref, w2_ref, b2_ref, o_hbm,
         xv, ov, in_sems, out_sems):
    for i in range(NCHUNK):
        pltpu.make_async_copy(
            x_hbm.at[pl.ds(i * CHUNK, CHUNK), :], xv.at[i], in_sems.at[i]
        ).start()
    for i in range(NCHUNK):
        pltpu.make_async_copy(
            x_hbm.at[pl.ds(i * CHUNK, CHUNK), :], xv.at[i], in_sems.at[i]
        ).wait()
        h = jnp.dot(xv[i], w0_ref[...], preferred_element_type=jnp.float32)
        h = jnp.maximum(h + b0_ref[...], 0.0)
        h = jnp.dot(h, w1_ref[...], preferred_element_type=jnp.float32)
        h = jnp.maximum(h + b1_ref[...], 0.0)
        o = jnp.dot(h, w2_ref[...], preferred_element_type=jnp.float32)
        ov[i] = o + b2_ref[...]
        pltpu.make_async_copy(
            ov.at[i], o_hbm.at[pl.ds(i * CHUNK, CHUNK), :], out_sems.at[i]
        ).start()
    for i in range(NCHUNK):
        pltpu.make_async_copy(
            ov.at[i], o_hbm.at[pl.ds(i * CHUNK, CHUNK), :], out_sems.at[i]
        ).wait()


@functools.partial(jax.jit, static_argnames=())
def kernel(x, weight, W0, b0, W1, b1, W2, b2, edge_index, batch):
    del weight, edge_index, batch
    b0r = b0.reshape(1, HID)
    b1r = b1.reshape(1, HID)
    b2r = b2.reshape(1, D_OUT)
    full = lambda: (0, 0)
    out = pl.pallas_call(
        _mlp,
        in_specs=[
            pl.BlockSpec(memory_space=pltpu.MemorySpace.HBM),
            pl.BlockSpec((D_IN, HID), full),
            pl.BlockSpec((1, HID), full),
            pl.BlockSpec((HID, HID), full),
            pl.BlockSpec((1, HID), full),
            pl.BlockSpec((HID, D_OUT), full),
            pl.BlockSpec((1, D_OUT), full),
        ],
        out_specs=pl.BlockSpec(memory_space=pltpu.MemorySpace.HBM),
        out_shape=jax.ShapeDtypeStruct((N, D_OUT), jnp.float32),
        scratch_shapes=[
            pltpu.VMEM((NCHUNK, CHUNK, D_IN), jnp.float32),
            pltpu.VMEM((NCHUNK, CHUNK, D_OUT), jnp.float32),
            pltpu.SemaphoreType.DMA((NCHUNK,)),
            pltpu.SemaphoreType.DMA((NCHUNK,)),
        ],
    )(x, W0, b0r, W1, b1r, W2, b2r)
    return out
